# baseline (speedup 1.0000x reference)
import functools

import jax
import jax.numpy as jnp
from jax import lax
from jax.experimental import pallas as pl
from jax.experimental.pallas import tpu as pltpu
from jax.experimental.pallas import tpu_sc as plsc

PRE_SEQ_LEN = 128
HIDDEN = 768
NUM_LAYERS = 12
ROW_DIM = NUM_LAYERS * 2 * HIDDEN
BATCH_N = 16
N_ROWS = BATCH_N * PRE_SEQ_LEN

_NC, _NS = 2, 16
_S = 16
_GB = 32 // _S
_W = ROW_DIM // _S
_RB = PRE_SEQ_LEN // _GB
_NG = N_ROWS // 16

_mesh = plsc.VectorSubcoreMesh(core_axis_name="c", subcore_axis_name="s")


@functools.partial(
    pl.kernel,
    mesh=_mesh,
    out_type=jax.ShapeDtypeStruct((N_ROWS, ROW_DIM), jnp.float32),
    scratch_types=[
        pltpu.VMEM((_RB, _W), jnp.float32),
        pltpu.VMEM((N_ROWS,), jnp.int32),
        pltpu.SemaphoreType.DMA,
        pltpu.SemaphoreType.DMA,
    ],
)
def _gather_kernel(idx_hbm, table_hbm, out_hbm, cache_v, idx_v, lsem, wsem):
    t = lax.axis_index("s") * _NC + lax.axis_index("c")
    g = t // _S
    s = t % _S
    lo = g * _RB
    coff = s * _W
    cl = pltpu.async_copy(table_hbm.at[pl.ds(lo, _RB), pl.ds(coff, _W)], cache_v, lsem)
    il = pltpu.async_copy(idx_hbm, idx_v, lsem)
    il.wait()
    cl.wait()

    goff = t * (_NG // 32)

    def step(gi0, cnt):
        gi = gi0 + goff
        gi = jnp.where(gi >= _NG, gi - _NG, gi)
        v = idx_v[pl.ds(gi * 16, 16)]
        for lane in range(16):
            r = v[lane]
            m = (r >> 6) == g

            @pl.when(m)
            def _():
                pltpu.async_copy(
                    cache_v.at[r & 63], out_hbm.at[gi * 16 + lane, pl.ds(coff, _W)], wsem
                )

            cnt = cnt + jnp.where(m, 1, 0)
        return cnt

    cnt = lax.fori_loop(0, _NG, step, jnp.int32(0))

    def drain(i, c):
        pltpu.make_async_copy(cache_v.at[0], out_hbm.at[0, pl.ds(coff, _W)], wsem).wait()
        return c

    lax.fori_loop(0, cnt, drain, jnp.int32(0))


def kernel(prefix, embedding_table):
    idx = prefix.reshape(N_ROWS)
    out = _gather_kernel(idx, embedding_table)
    return out.reshape(BATCH_N, PRE_SEQ_LEN, ROW_DIM)


# speedup vs baseline: 1.0149x; 1.0149x over previous
import functools

import jax
import jax.numpy as jnp
from jax import lax
from jax.experimental import pallas as pl
from jax.experimental.pallas import tpu as pltpu
from jax.experimental.pallas import tpu_sc as plsc

PRE_SEQ_LEN = 128
HIDDEN = 768
NUM_LAYERS = 12
ROW_DIM = NUM_LAYERS * 2 * HIDDEN
BATCH_N = 16
N_ROWS = BATCH_N * PRE_SEQ_LEN

_NC, _NS = 2, 16
_S = 16
_GB = 32 // _S
_W = ROW_DIM // _S
_RB = PRE_SEQ_LEN // _GB
_NG = N_ROWS // 16

_mesh = plsc.VectorSubcoreMesh(core_axis_name="c", subcore_axis_name="s")


@functools.partial(
    pl.kernel,
    mesh=_mesh,
    out_type=jax.ShapeDtypeStruct((N_ROWS, ROW_DIM), jnp.float32),
    scratch_types=[
        pltpu.VMEM((_RB, _W), jnp.float32),
        pltpu.VMEM((N_ROWS,), jnp.int32),
        pltpu.SemaphoreType.DMA,
        pltpu.SemaphoreType.DMA,
    ],
)
def _gather_kernel(idx_hbm, table_hbm, out_hbm, cache_v, idx_v, lsem, wsem):
    t = lax.axis_index("s") * _NC + lax.axis_index("c")
    g = t // _S
    s = t % _S
    lo = g * _RB
    coff = s * _W
    cl = pltpu.async_copy(table_hbm.at[pl.ds(lo, _RB), pl.ds(coff, _W)], cache_v, lsem)
    il = pltpu.async_copy(idx_hbm, idx_v, lsem)
    il.wait()
    cl.wait()

    def step(gi, cnt):
        v = idx_v[pl.ds(gi * 16, 16)]
        for lane in range(16):
            r = v[lane]
            m = (r >> 6) == g

            @pl.when(m)
            def _():
                pltpu.async_copy(
                    cache_v.at[r & 63], out_hbm.at[gi * 16 + lane, pl.ds(coff, _W)], wsem
                )

            cnt = cnt + jnp.where(m, 1, 0)
        return cnt

    cnt = lax.fori_loop(0, _NG, step, jnp.int32(0))

    def drain(i, c):
        pltpu.make_async_copy(cache_v.at[0], out_hbm.at[0, pl.ds(coff, _W)], wsem).wait()
        return c

    lax.fori_loop(0, cnt, drain, jnp.int32(0))


def kernel(prefix, embedding_table):
    idx = prefix.reshape(N_ROWS)
    out = _gather_kernel(idx, embedding_table)
    return out.reshape(BATCH_N, PRE_SEQ_LEN, ROW_DIM)
